# Initial kernel scaffold; baseline (speedup 1.0000x reference)
import jax
import jax.numpy as jnp
from jax.experimental import pallas as pl


def kernel(h, edge_index, W0, b0, W1, b1, W2, b2, W3, b3):
    raise NotImplementedError("write your pallas kernel here")


# speedup vs baseline: 4.3982x; 4.3982x over previous
import functools

import jax
import jax.numpy as jnp
from jax import lax
from jax.experimental import pallas as pl
from jax.experimental.pallas import tpu as pltpu
from jax.experimental.pallas import tpu_sc as plsc

N_NODES = 10000
D = 128
NC = 2
NS = 16
NW = NC * NS
BK = 128
NBLK = 79
ROWS_PER_TILE = 640
NROWS = NS * ROWS_PER_TILE
ZCH = 128


def _sc_agg_body(h_hbm, src_hbm, dst_hbm, zrows_hbm, out_hbm,
                 src_v, dst_v, gbuf, agg_sh, sem):
    c = lax.axis_index("c")
    s = lax.axis_index("s")
    wid = s * NC + c

    pltpu.sync_copy(src_hbm.at[wid], src_v)
    pltpu.sync_copy(dst_hbm.at[wid], dst_v)

    row0 = s * ROWS_PER_TILE

    @pl.loop(0, ROWS_PER_TILE // ZCH)
    def _zero(k):
        pltpu.sync_copy(zrows_hbm, agg_sh.at[pl.ds(row0 + k * ZCH, ZCH)])

    plsc.subcore_barrier()

    @pl.loop(0, NBLK)
    def _edges(j):
        pltpu.async_copy(h_hbm.at[src_v.at[j]], gbuf, sem).wait()
        pltpu.sync_copy(gbuf, agg_sh.at[dst_v.at[j]], add=True)

    plsc.subcore_barrier()

    pltpu.sync_copy(agg_sh.at[pl.ds(row0, ROWS_PER_TILE)],
                    out_hbm.at[c, pl.ds(row0, ROWS_PER_TILE)])


@jax.jit
def _sc_agg(h, src_r, dst_r, zrows):
    mesh = plsc.VectorSubcoreMesh(core_axis_name="c", subcore_axis_name="s")
    return pl.kernel(
        _sc_agg_body,
        out_type=jax.ShapeDtypeStruct((NC, NROWS, D), jnp.float32),
        mesh=mesh,
        scratch_types=[
            pltpu.VMEM((NBLK, BK), jnp.int32),
            pltpu.VMEM((NBLK, BK), jnp.int32),
            pltpu.VMEM((BK, D), jnp.float32),
            pltpu.VMEM_SHARED((NROWS, D), jnp.float32),
            pltpu.SemaphoreType.DMA,
        ],
    )(h, src_r, dst_r, zrows)


def _tc_update_body(h_ref, p0_ref, p1_ref, w_ref, b_ref, o_ref):
    x = h_ref[...] + p0_ref[0] + p1_ref[0]
    o_ref[...] = (
        jnp.dot(x, w_ref[...], preferred_element_type=jnp.float32,
                precision=lax.Precision.HIGHEST)
        + b_ref[...]
    )


@jax.jit
def _tc_update(h, parts, W, b2d):
    rb = 1000
    grid = (N_NODES // rb,)
    return pl.pallas_call(
        _tc_update_body,
        grid=grid,
        in_specs=[
            pl.BlockSpec((rb, D), lambda i: (i, 0)),
            pl.BlockSpec((1, rb, D), lambda i: (0, i, 0)),
            pl.BlockSpec((1, rb, D), lambda i: (1, i, 0)),
            pl.BlockSpec((D, D), lambda i: (0, 0)),
            pl.BlockSpec((1, D), lambda i: (0, 0)),
        ],
        out_specs=pl.BlockSpec((rb, D), lambda i: (i, 0)),
        out_shape=jax.ShapeDtypeStruct((N_NODES, D), jnp.float32),
    )(h, parts, parts, W, b2d)


def kernel(h, edge_index, W0, b0, W1, b1, W2, b2, W3, b3):
    src = edge_index[0].astype(jnp.int32)
    dst = edge_index[1].astype(jnp.int32)
    n_edges = src.shape[0]
    total = NW * NBLK * BK
    pad = total - n_edges
    src_r = jnp.concatenate(
        [src, jnp.zeros((pad,), jnp.int32)]).reshape(NW, NBLK, BK)
    dst_r = jnp.concatenate(
        [dst, jnp.full((pad,), N_NODES, jnp.int32)]).reshape(NW, NBLK, BK)
    zrows = jnp.zeros((ZCH, D), jnp.float32)

    params = [(W0, b0), (W1, b1), (W2, b2), (W3, b3)]
    for W, b in params:
        parts = _sc_agg(h, src_r, dst_r, zrows)
        h = _tc_update(h, parts, W, b.reshape(1, D))
    return h
